# Initial kernel scaffold; baseline (speedup 1.0000x reference)
import jax
import jax.numpy as jnp
from jax.experimental import pallas as pl


def kernel(x, mask_sites):
    raise NotImplementedError("write your pallas kernel here")


# speedup vs baseline: 4.9964x; 4.9964x over previous
import functools

import jax
import jax.numpy as jnp
from jax.experimental import pallas as pl

C, H, W = 3, 4096, 4096
N_SITES = 1228
PAD_SITES = 1280
BH = 256


def _mask_kernel(sites_ref, mask_ref):
    iota = jax.lax.broadcasted_iota(jnp.int32, (1, W), 1)
    acc = jnp.zeros((1, W), dtype=jnp.float32)
    for j in range(PAD_SITES // 128):
        row = sites_ref[j, :].reshape(128, 1)
        hit = jnp.any(row == iota, axis=0, keepdims=True)
        acc = jnp.maximum(acc, hit.astype(jnp.float32))
    mask_ref[...] = acc


def _apply_kernel(mask_ref, x_ref, out_ref):
    c = pl.program_id(0)
    xb = x_ref[...]
    m = mask_ref[...].reshape(1, 1, W)
    cond = (m > 0.0) & (c == 0)
    out_ref[...] = jnp.where(cond, 1.0 - xb, xb)


@jax.jit
def kernel(x, mask_sites):
    sites = mask_sites.astype(jnp.int32)
    sites = jnp.pad(sites, (0, PAD_SITES - N_SITES), constant_values=jnp.int32(1 << 30))
    sites = sites.reshape(PAD_SITES // 128, 128)

    mask = pl.pallas_call(
        _mask_kernel,
        out_shape=jax.ShapeDtypeStruct((1, W), jnp.float32),
    )(sites)

    out = pl.pallas_call(
        _apply_kernel,
        grid=(C, H // BH),
        in_specs=[
            pl.BlockSpec((1, W), lambda c, h: (0, 0)),
            pl.BlockSpec((1, BH, W), lambda c, h: (c, h, 0)),
        ],
        out_specs=pl.BlockSpec((1, BH, W), lambda c, h: (c, h, 0)),
        out_shape=jax.ShapeDtypeStruct((C, H, W), jnp.float32),
    )(mask, x)
    return out


# speedup vs baseline: 5.0994x; 1.0206x over previous
import jax
import jax.numpy as jnp
from jax.experimental import pallas as pl
from jax.experimental.pallas import tpu as pltpu

C, H, W = 3, 4096, 4096
N_SITES = 1228
PAD_SITES = 1280
BH = 512


def _fused_kernel(sites_ref, x_ref, out_ref, mask_ref):
    c = pl.program_id(0)
    h = pl.program_id(1)

    @pl.when((c == 0) & (h == 0))
    def _build_mask():
        iota = jax.lax.broadcasted_iota(jnp.int32, (1, W), 1)
        acc = jnp.zeros((1, W), dtype=jnp.float32)
        for j in range(PAD_SITES // 128):
            row = sites_ref[j, :].reshape(128, 1)
            hit = jnp.any(row == iota, axis=0, keepdims=True)
            acc = jnp.maximum(acc, hit.astype(jnp.float32))
        mask_ref[...] = acc

    xb = x_ref[...]

    @pl.when(c < C - 1)
    def _copy():
        out_ref[...] = xb

    @pl.when(c == C - 1)
    def _flip():
        m = mask_ref[...].reshape(1, 1, W)
        out_ref[...] = jnp.where(m > 0.0, 1.0 - xb, xb)


@jax.jit
def kernel(x, mask_sites):
    sites = mask_sites.astype(jnp.int32)
    sites = jnp.pad(sites, (0, PAD_SITES - N_SITES), constant_values=jnp.int32(1 << 30))
    sites = sites.reshape(PAD_SITES // 128, 128)

    out = pl.pallas_call(
        _fused_kernel,
        grid=(C, H // BH),
        in_specs=[
            pl.BlockSpec((PAD_SITES // 128, 128), lambda c, h: (0, 0)),
            pl.BlockSpec((1, BH, W), lambda c, h: ((c + 1) % C, h, 0)),
        ],
        out_specs=pl.BlockSpec((1, BH, W), lambda c, h: ((c + 1) % C, h, 0)),
        out_shape=jax.ShapeDtypeStruct((C, H, W), jnp.float32),
        scratch_shapes=[pltpu.VMEM((1, W), jnp.float32)],
    )(sites, x)
    return out
